# baseline (speedup 1.0000x reference)
import functools

import jax
import jax.numpy as jnp
from jax import lax
from jax.experimental import pallas as pl
from jax.experimental.pallas import tpu as pltpu
from jax.experimental.pallas import tpu_sc as plsc

D = 64
B = 16384
L = 16
NC = 2
NS = 16
NW = NC * NS
ROWS_PER_W = B // NW
CHUNK = 128
NCHUNK = ROWS_PER_W // CHUNK
GROUPS = CHUNK // L
EPS = 1e-6


def _sqrt16(x):
    i = plsc.bitcast(x, jnp.int32)
    y = plsc.bitcast((i >> 1) + jnp.int32(0x1FBD1DF5), jnp.float32)
    half = jnp.float32(0.5)
    y = half * (y + x / y)
    y = half * (y + x / y)
    y = half * (y + x / y)
    return y


def _body(u_idx_hbm, i_idx_hbm, u_tab, i_tab, out_hbm,
          idx_u_v, idx_i_v, rows_u, rows_i, out_v, sem_u, sem_i):
    wid = lax.axis_index("s") * NC + lax.axis_index("c")
    blk = wid * NCHUNK
    pltpu.sync_copy(u_idx_hbm.at[pl.ds(blk, NCHUNK)], idx_u_v)
    pltpu.sync_copy(i_idx_hbm.at[pl.ds(blk, NCHUNK)], idx_i_v)
    lanes = lax.iota(jnp.int32, L)

    for c in range(NCHUNK):
        cu = pltpu.async_copy(u_tab.at[idx_u_v.at[c]], rows_u, sem_u)
        ci = pltpu.async_copy(i_tab.at[idx_i_v.at[c]], rows_i, sem_i)
        cu.wait()
        ci.wait()

        def group(g, _, c=c):
            ridx = g * L + lanes
            acc = jnp.zeros((L,), jnp.float32)
            for j in range(D):
                jv = jnp.full((L,), j, jnp.int32)
                u = plsc.load_gather(rows_u, [ridx, jv])
                v = plsc.load_gather(rows_i, [ridx, jv])
                d = (u - v) + jnp.float32(EPS)
                acc = acc + d * d
            out_v[pl.ds(c * CHUNK + g * L, L)] = _sqrt16(acc)
            return 0

        lax.fori_loop(0, GROUPS, group, 0)

    base = wid * ROWS_PER_W
    pltpu.sync_copy(out_v, out_hbm.at[pl.ds(base, ROWS_PER_W)])


@functools.partial(
    pl.kernel,
    mesh=plsc.VectorSubcoreMesh(core_axis_name="c", subcore_axis_name="s"),
    out_type=jax.ShapeDtypeStruct((B,), jnp.float32),
    compiler_params=pltpu.CompilerParams(
        needs_layout_passes=False, use_tc_tiling_on_sc=False
    ),
    scratch_types=[
        pltpu.VMEM((NCHUNK, CHUNK), jnp.int32),
        pltpu.VMEM((NCHUNK, CHUNK), jnp.int32),
        pltpu.VMEM((CHUNK, D), jnp.float32),
        pltpu.VMEM((CHUNK, D), jnp.float32),
        pltpu.VMEM((ROWS_PER_W,), jnp.float32),
        pltpu.SemaphoreType.DMA,
        pltpu.SemaphoreType.DMA,
    ],
)
def _cml_dist(u_idx, i_idx, u_tab, i_tab, out, *scratch):
    _body(u_idx, i_idx, u_tab, i_tab, out, *scratch)


def kernel(U, I, UEmb_weight, IEmb_weight):
    U2 = U.reshape(NW * NCHUNK, CHUNK)
    I2 = I.reshape(NW * NCHUNK, CHUNK)
    return _cml_dist(U2, I2, UEmb_weight, IEmb_weight)
